# baseline (speedup 1.0000x reference)
import functools

import jax
import jax.numpy as jnp
from jax import lax
from jax.experimental import pallas as pl
from jax.experimental.pallas import tpu as pltpu
from jax.experimental.pallas import tpu_sc as plsc

NC = 2
NS = 16
L = 16
NW = NC * NS

BATCH = 16384
EMBED = 64
WIDE = 2 * EMBED
CHUNK = 128
B_PER_W = BATCH // NW
N_CHUNKS = B_PER_W // CHUNK
GPC = CHUNK // L


def _mf_body(u_id_hbm, i_id_hbm, user_emb_hbm, user_bias_hbm,
             item_emb_hbm, item_bias_hbm, mean_hbm, out_hbm,
             uidx_v, iidx_v, ugidx_v, igidx_v, urows_v, irows_v,
             ub_v, ib_v, out_v, mean_v, bias_sem, row_sem):
    wid = lax.axis_index("s") * NC + lax.axis_index("c")
    row0 = wid * N_CHUNKS

    pltpu.sync_copy(u_id_hbm.at[pl.ds(row0, N_CHUNKS)], uidx_v)
    pltpu.sync_copy(i_id_hbm.at[pl.ds(row0, N_CHUNKS)], iidx_v)
    pltpu.sync_copy(mean_hbm, mean_v)

    for c in range(N_CHUNKS):
        for s in range(CHUNK // L):
            sl = pl.ds(s * L, L)
            ugidx_v[c, sl] = lax.shift_right_logical(uidx_v[c, sl], 1)
            igidx_v[c, sl] = lax.shift_right_logical(iidx_v[c, sl], 1)

    bias_copies = []
    for c in range(N_CHUNKS):
        sl = pl.ds(c * CHUNK, CHUNK)
        bias_copies.append(pltpu.async_copy(
            user_bias_hbm.at[uidx_v.at[c]], ub_v.at[sl], bias_sem))
        bias_copies.append(pltpu.async_copy(
            item_bias_hbm.at[iidx_v.at[c]], ib_v.at[sl], bias_sem))

    def fire(c):
        b = c % 2
        sl = pl.ds(b * CHUNK, CHUNK)
        return (pltpu.async_copy(
                    user_emb_hbm.at[ugidx_v.at[c]], urows_v.at[sl], row_sem),
                pltpu.async_copy(
                    item_emb_hbm.at[igidx_v.at[c]], irows_v.at[sl], row_sem))

    mean_vec = mean_v[...]

    def make_group_body(c):
        b = c % 2

        def group_body(g, _):
            rows = b * CHUNK + g * L + lax.iota(jnp.int32, L)
            upar = (uidx_v[c, pl.ds(g * L, L)] & 1) * EMBED
            ipar = (iidx_v[c, pl.ds(g * L, L)] & 1) * EMBED
            acc = jnp.zeros((L,), jnp.float32)
            for j in range(EMBED):
                uj = plsc.load_gather(urows_v, [rows, upar + j])
                ij = plsc.load_gather(irows_v, [rows, ipar + j])
                acc = acc + uj * ij
            out_v[pl.ds((c * GPC + g) * L, L)] = acc
            return 0

        return group_body

    pending = fire(0)
    for c in range(N_CHUNKS):
        nxt = fire(c + 1) if c + 1 < N_CHUNKS else None
        for cp in pending:
            cp.wait()
        lax.fori_loop(0, GPC, make_group_body(c), 0)
        pending = nxt

    for cp in bias_copies:
        cp.wait()
    for g in range(N_CHUNKS * GPC):
        sl = pl.ds(g * L, L)
        out_v[sl] = out_v[sl] + ub_v[sl] + ib_v[sl] + mean_vec

    pltpu.sync_copy(out_v, out_hbm.at[pl.ds(wid * B_PER_W, B_PER_W)])


@functools.partial(jax.jit, static_argnames=())
def kernel(u_id, i_id, user_emb, user_bias, item_emb, item_bias, mean):
    mesh = plsc.VectorSubcoreMesh(
        core_axis_name="c", subcore_axis_name="s",
        num_cores=NC, num_subcores=NS)
    f = pl.kernel(
        _mf_body,
        out_type=jax.ShapeDtypeStruct((BATCH,), jnp.float32),
        mesh=mesh,
        compiler_params=pltpu.CompilerParams(
            needs_layout_passes=False, use_tc_tiling_on_sc=True),
        scratch_types=[
            pltpu.VMEM((N_CHUNKS, CHUNK), jnp.int32),
            pltpu.VMEM((N_CHUNKS, CHUNK), jnp.int32),
            pltpu.VMEM((N_CHUNKS, CHUNK), jnp.int32),
            pltpu.VMEM((N_CHUNKS, CHUNK), jnp.int32),
            pltpu.VMEM((2 * CHUNK, WIDE), jnp.float32),
            pltpu.VMEM((2 * CHUNK, WIDE), jnp.float32),
            pltpu.VMEM((B_PER_W,), jnp.float32),
            pltpu.VMEM((B_PER_W,), jnp.float32),
            pltpu.VMEM((B_PER_W,), jnp.float32),
            pltpu.VMEM((L,), jnp.float32),
            pltpu.SemaphoreType.DMA,
            pltpu.SemaphoreType.DMA,
        ],
    )
    u2 = u_id.reshape(BATCH // CHUNK, CHUNK).astype(jnp.int32)
    i2 = i_id.reshape(BATCH // CHUNK, CHUNK).astype(jnp.int32)
    mean16 = jnp.broadcast_to(mean, (L,))
    return f(u2, i2, user_emb.reshape(-1, WIDE), user_bias.reshape(-1),
             item_emb.reshape(-1, WIDE), item_bias.reshape(-1), mean16)
